# Initial kernel scaffold; baseline (speedup 1.0000x reference)
import jax
import jax.numpy as jnp
from jax.experimental import pallas as pl


def kernel(hours, days, months, hour_table, day_table, month_table):
    raise NotImplementedError("write your pallas kernel here")


# speedup vs baseline: 1.8622x; 1.8622x over previous
import functools

import jax
import jax.numpy as jnp
from jax import lax
from jax.experimental import pallas as pl
from jax.experimental.pallas import tpu as pltpu
from jax.experimental.pallas import tpu_sc as plsc

B, S, D = 16384, 200, 32
N = B * S
NC, NS = 2, 16
NW = NC * NS
PER_W = N // NW
GB = 128
K = 1024
NGB = K // GB
NBLK = PER_W // K
ROWS_PER_BLK = K // GB


def _body(hours_ref, days_ref, months_ref, ht_ref, dt_ref, mt_ref, out_ref,
          h_idx, d_idx, m_idx, obuf, sem):
    wid = lax.axis_index("s") * NC + lax.axis_index("c")
    w_row0 = wid * (PER_W // GB)

    def block(b, carry):
        row0 = w_row0 + b * ROWS_PER_BLK
        base = row0 * GB
        pltpu.sync_copy(hours_ref.at[pl.ds(row0, ROWS_PER_BLK)], h_idx)
        pltpu.sync_copy(days_ref.at[pl.ds(row0, ROWS_PER_BLK)], d_idx)
        pltpu.sync_copy(months_ref.at[pl.ds(row0, ROWS_PER_BLK)], m_idx)

        descs = [
            pltpu.async_copy(ht_ref.at[h_idx.at[j]],
                             obuf.at[pl.ds(j * GB, GB)], sem)
            for j in range(NGB)
        ]
        for d in descs:
            d.wait()
        descs = [
            pltpu.async_copy(dt_ref.at[d_idx.at[j]],
                             obuf.at[pl.ds(j * GB, GB)], sem, add=True)
            for j in range(NGB)
        ]
        for d in descs:
            d.wait()
        descs = [
            pltpu.async_copy(mt_ref.at[m_idx.at[j]],
                             obuf.at[pl.ds(j * GB, GB)], sem, add=True)
            for j in range(NGB)
        ]
        for d in descs:
            d.wait()

        pltpu.sync_copy(obuf, out_ref.at[pl.ds(base, K)])
        return carry

    lax.fori_loop(0, NBLK, block, 0)


@functools.partial(jax.jit, static_argnames=())
def _run(hours2, days2, months2, hour_table, day_table, month_table):
    mesh = plsc.VectorSubcoreMesh(core_axis_name="c", subcore_axis_name="s")
    kern = pl.kernel(
        _body,
        out_type=jax.ShapeDtypeStruct((N, D), jnp.float32),
        mesh=mesh,
        scratch_types=[
            pltpu.VMEM((ROWS_PER_BLK, GB), jnp.int32),
            pltpu.VMEM((ROWS_PER_BLK, GB), jnp.int32),
            pltpu.VMEM((ROWS_PER_BLK, GB), jnp.int32),
            pltpu.VMEM((K, D), jnp.float32),
            pltpu.SemaphoreType.DMA,
        ],
        compiler_params=pltpu.CompilerParams(use_tc_tiling_on_sc=False),
    )
    return kern(hours2, days2, months2, hour_table, day_table, month_table)


def kernel(hours, days, months, hour_table, day_table, month_table):
    hours2 = hours.astype(jnp.int32).reshape(N // GB, GB)
    days2 = days.astype(jnp.int32).reshape(N // GB, GB)
    months2 = months.astype(jnp.int32).reshape(N // GB, GB)
    out = _run(hours2, days2, months2, hour_table, day_table, month_table)
    return out.reshape(B, S, D)
